# baseline (speedup 1.0000x reference)
import functools

import jax
import jax.numpy as jnp
from jax import lax
from jax.experimental import pallas as pl
from jax.experimental.pallas import tpu as pltpu
from jax.experimental.pallas import tpu_sc as plsc

BATCH = 4096
HIST = 50
DIM = 64
NUM_EMB = 1000000
NUM_WORKERS = 32
NBB = BATCH // 128
NCHUNK = HIST * NBB
PER_WORKER = NCHUNK // NUM_WORKERS

_mesh = plsc.VectorSubcoreMesh(core_axis_name="c", subcore_axis_name="s")


@functools.partial(
    pl.kernel,
    mesh=_mesh,
    out_type=jax.ShapeDtypeStruct((HIST, DIM, BATCH), jnp.float32),
    scratch_types=[
        pltpu.VMEM((2, 128), jnp.int32),
        pltpu.VMEM((2, 128), jnp.int32),
        pltpu.VMEM((2, 128, 128), jnp.float32),
        pltpu.VMEM((2, DIM, 128), jnp.float32),
        pltpu.SemaphoreType.DMA,
        pltpu.SemaphoreType.DMA,
        pltpu.SemaphoreType.DMA,
        pltpu.SemaphoreType.DMA,
    ],
    compiler_params=pltpu.CompilerParams(use_tc_tiling_on_sc=True,
                                         needs_layout_passes=False),
)
def _emb_lookup(idx_hbm, table_hbm, out_hbm, idx_v, plist_v, chunk_v, xout_v,
                sem0, sem1, osem0, osem1):
    wid = lax.axis_index("s") * 2 + lax.axis_index("c")
    base = wid * PER_WORKER
    sems = (sem0, sem1)
    osems = (osem0, osem1)

    def prep_and_fire(k, slot):
        ci = base + k
        h = ci // NBB
        bb = ci % NBB
        pltpu.sync_copy(idx_hbm.at[h, pl.ds(bb * 128, 128)], idx_v.at[slot])
        for g in range(8):
            ids = idx_v[slot, pl.ds(16 * g, 16)]
            plist_v[slot, pl.ds(16 * g, 16)] = lax.shift_right_logical(ids, 1)
        pltpu.async_copy(table_hbm.at[plist_v.at[slot]], chunk_v.at[slot],
                         sems[slot])

    def consume(k, slot):
        ci = base + k
        h = ci // NBB
        bb = ci % NBB
        pltpu.make_async_copy(table_hbm.at[plist_v.at[slot]],
                              chunk_v.at[slot], sems[slot]).wait()

        @pl.when(k >= 2)
        def _():
            pltpu.make_async_copy(out_hbm.at[0, :, pl.ds(0, 128)],
                                  xout_v.at[slot], osems[slot]).wait()

        iota16 = jax.lax.iota(jnp.int32, 16)
        rows = [iota16 + 16 * g for g in range(8)]
        halves = tuple((idx_v[slot, pl.ds(16 * g, 16)] & 1) * DIM
                       for g in range(8))

        def tr_body(f0, carry):
            colf = (f0 + iota16) & (DIM - 1)
            for g in range(8):
                vals = plsc.load_gather(chunk_v.at[slot],
                                        [rows[g], carry[g] + colf])
                plsc.store_scatter(xout_v.at[slot], [colf, rows[g]], vals)
            return carry

        lax.fori_loop(0, DIM, tr_body, halves, unroll=2)
        pltpu.async_copy(xout_v.at[slot], out_hbm.at[h, :, pl.ds(bb * 128, 128)],
                         osems[slot])

    prep_and_fire(0, 0)

    def grp(gi, carry):
        for b in range(2):
            k = 2 * gi + b

            @pl.when(k + 1 < PER_WORKER)
            def _():
                prep_and_fire(k + 1, 1 - b)

            consume(k, b)
        return carry

    lax.fori_loop(0, PER_WORKER // 2, grp, 0)

    for b in range(2):
        pltpu.make_async_copy(out_hbm.at[0, :, pl.ds(0, 128)],
                              xout_v.at[b], osems[b]).wait()


def kernel(inputs, weight):
    idx_t = jnp.transpose(inputs).astype(jnp.int32)
    table = jnp.reshape(weight, (NUM_EMB // 2, 128))
    out = _emb_lookup(idx_t, table)
    return jnp.transpose(out, (2, 0, 1))


# speedup vs baseline: 1.0987x; 1.0987x over previous
import functools

import jax
import jax.numpy as jnp
from jax import lax
from jax.experimental import pallas as pl
from jax.experimental.pallas import tpu as pltpu
from jax.experimental.pallas import tpu_sc as plsc

BATCH = 4096
HIST = 50
DIM = 64
NUM_EMB = 1000000
NUM_WORKERS = 32
NBB = BATCH // 128
NCHUNK = HIST * NBB
PER_WORKER = NCHUNK // NUM_WORKERS

_mesh = plsc.VectorSubcoreMesh(core_axis_name="c", subcore_axis_name="s")


@functools.partial(
    pl.kernel,
    mesh=_mesh,
    out_type=jax.ShapeDtypeStruct((HIST, DIM, BATCH), jnp.float32),
    scratch_types=[
        pltpu.VMEM((2, 128), jnp.int32),
        pltpu.VMEM((2, 128, 128), jnp.float32),
        pltpu.VMEM((2, DIM, 128), jnp.float32),
        pltpu.SemaphoreType.DMA,
        pltpu.SemaphoreType.DMA,
        pltpu.SemaphoreType.DMA,
        pltpu.SemaphoreType.DMA,
    ],
    compiler_params=pltpu.CompilerParams(use_tc_tiling_on_sc=True,
                                         needs_layout_passes=False),
)
def _emb_lookup(idx_hbm, table_hbm, out_hbm, idx_v, chunk_v, xout_v,
                sem0, sem1, osem0, osem1):
    wid = lax.axis_index("s") * 2 + lax.axis_index("c")
    base = wid * PER_WORKER
    sems = (sem0, sem1)
    osems = (osem0, osem1)

    def prep_and_fire(k, slot):
        ci = base + k
        h = ci // NBB
        bb = ci % NBB
        pltpu.sync_copy(idx_hbm.at[h, pl.ds(bb * 128, 128)], idx_v.at[slot])
        pltpu.async_copy(table_hbm.at[idx_v.at[slot]], chunk_v.at[slot],
                         sems[slot])

    def consume(k, slot):
        ci = base + k
        h = ci // NBB
        bb = ci % NBB
        pltpu.make_async_copy(table_hbm.at[idx_v.at[slot]],
                              chunk_v.at[slot], sems[slot]).wait()

        @pl.when(k >= 2)
        def _():
            pltpu.make_async_copy(out_hbm.at[0, :, pl.ds(0, 128)],
                                  xout_v.at[slot], osems[slot]).wait()

        iota16 = jax.lax.iota(jnp.int32, 16)
        rows = [iota16 + 16 * g for g in range(8)]

        def tr_body(f0, carry):
            colf = (f0 + iota16) & (DIM - 1)
            for g in range(8):
                vals = plsc.load_gather(chunk_v.at[slot], [rows[g], colf])
                plsc.store_scatter(xout_v.at[slot], [colf, rows[g]], vals)
            return carry

        lax.fori_loop(0, DIM, tr_body, 0, unroll=2)
        pltpu.async_copy(xout_v.at[slot], out_hbm.at[h, :, pl.ds(bb * 128, 128)],
                         osems[slot])

    prep_and_fire(0, 0)

    def grp(gi, carry):
        for b in range(2):
            k = 2 * gi + b

            @pl.when(k + 1 < PER_WORKER)
            def _():
                prep_and_fire(k + 1, 1 - b)

            consume(k, b)
        return carry

    lax.fori_loop(0, PER_WORKER // 2, grp, 0)

    for b in range(2):
        pltpu.make_async_copy(out_hbm.at[0, :, pl.ds(0, 128)],
                              xout_v.at[b], osems[b]).wait()


def kernel(inputs, weight):
    idx_t = jnp.transpose(inputs).astype(jnp.int32)
    table = jnp.pad(weight, ((0, 0), (0, 128 - DIM)))
    out = _emb_lookup(idx_t, table)
    return jnp.transpose(out, (2, 0, 1))
